# Initial kernel scaffold; baseline (speedup 1.0000x reference)
import jax
import jax.numpy as jnp
from jax.experimental import pallas as pl


def kernel(x1, edge_index1, batch1, cell, x2, edge_index2, batch2, W1, b1, W2, b2, W3, b3, Wg1, bg1, Wg2, bg2, Wr1, br1, Wr2, br2, Wr3, br3, Wf1, bf1, Wf2, bf2, Wo, bo):
    raise NotImplementedError("write your pallas kernel here")


# speedup vs baseline: 6.7981x; 6.7981x over previous
import functools

import jax
import jax.numpy as jnp
from jax import lax
from jax.experimental import pallas as pl
from jax.experimental.pallas import tpu as pltpu
from jax.experimental.pallas import tpu_sc as plsc

N = 10000
NP = 10240
E = 320000
B = 512
NC = 2
NS = 16
CH = 80


@functools.lru_cache(maxsize=None)
def _make_sc_scatter_add(n, w):
    ept = E // (NC * NS)
    iters = ept // CH
    rps = NP // NS
    mesh = plsc.VectorSubcoreMesh(core_axis_name="c", subcore_axis_name="s")

    @functools.partial(
        pl.kernel, mesh=mesh,
        compiler_params=pltpu.CompilerParams(use_tc_tiling_on_sc=False),
        out_type=jax.ShapeDtypeStruct((NC, NP, w), jnp.float32),
        scratch_types=[
            pltpu.VMEM((CH,), jnp.int32),
            pltpu.VMEM((CH,), jnp.int32),
            pltpu.VMEM((CH, w), jnp.float32),
            pltpu.VMEM_SHARED((NP, w), jnp.float32),
            pltpu.SemaphoreType.DMA,
        ],
    )
    def k(u_hbm, s_hbm, d_hbm, z_hbm, out_hbm, sv, dv, rows, acc, sem):
        cid = lax.axis_index("c")
        sid = lax.axis_index("s")
        pltpu.sync_copy(z_hbm.at[pl.ds(sid * rps, rps)],
                        acc.at[pl.ds(sid * rps, rps)])
        plsc.subcore_barrier()
        base = (cid * NS + sid) * ept

        def body(i, carry):
            off = base + i * CH
            pltpu.sync_copy(s_hbm.at[pl.ds(off, CH)], sv)
            pltpu.sync_copy(d_hbm.at[pl.ds(off, CH)], dv)
            pltpu.async_copy(u_hbm.at[sv], rows, sem).wait()
            pltpu.sync_copy(rows, acc.at[dv], add=True)
            return carry

        lax.fori_loop(0, iters, body, 0)
        plsc.subcore_barrier()
        pltpu.sync_copy(acc.at[pl.ds(sid * rps, rps)],
                        out_hbm.at[cid, pl.ds(sid * rps, rps)])

    return k


def _sc_scatter_add(u, s_idx, d_idx):
    n, w = u.shape
    zero = jnp.zeros((NP, w), jnp.float32)
    out = _make_sc_scatter_add(n, w)(u, s_idx, d_idx, zero)
    return out[:, :N, :]


_RB = 1000


def _mm_scale(h, w, dinv):
    din = h.shape[1]
    wp = w.shape[1]

    def body(h_ref, w_ref, dv_ref, o_ref):
        o_ref[...] = dv_ref[...] * jnp.dot(
            h_ref[...], w_ref[...], preferred_element_type=jnp.float32)

    return pl.pallas_call(
        body,
        grid=(N // _RB,),
        in_specs=[
            pl.BlockSpec((_RB, din), lambda i: (i, 0)),
            pl.BlockSpec((din, wp), lambda i: (0, 0)),
            pl.BlockSpec((_RB, 1), lambda i: (i, 0)),
        ],
        out_specs=pl.BlockSpec((_RB, wp), lambda i: (i, 0)),
        out_shape=jax.ShapeDtypeStruct((N, wp), jnp.float32),
    )(h, w, dinv)


def _combine(a0, a1, u, dinv, b):
    wp = u.shape[1]

    def body(a0_ref, a1_ref, u_ref, dv_ref, b_ref, o_ref):
        s = a0_ref[...] + a1_ref[...] + u_ref[...]
        o_ref[...] = jnp.maximum(dv_ref[...] * s + b_ref[...], 0.0)

    return pl.pallas_call(
        body,
        grid=(N // _RB,),
        in_specs=[
            pl.BlockSpec((_RB, wp), lambda i: (i, 0)),
            pl.BlockSpec((_RB, wp), lambda i: (i, 0)),
            pl.BlockSpec((_RB, wp), lambda i: (i, 0)),
            pl.BlockSpec((_RB, 1), lambda i: (i, 0)),
            pl.BlockSpec((1, wp), lambda i: (0, 0)),
        ],
        out_specs=pl.BlockSpec((_RB, wp), lambda i: (i, 0)),
        out_shape=jax.ShapeDtypeStruct((N, wp), jnp.float32),
    )(a0, a1, u, dinv, b)


def _dinv_from_deg(d0, d1):
    def body(d0_ref, d1_ref, o_ref):
        deg = 1.0 + d0_ref[:, 0:1] + d1_ref[:, 0:1]
        o_ref[...] = lax.rsqrt(deg)

    return pl.pallas_call(
        body,
        out_shape=jax.ShapeDtypeStruct((N, 1), jnp.float32),
    )(d0, d1)


def _segment_max(h, batch):
    wp = h.shape[1]

    def body(batch_ref, h_ref, o_ref):
        o_ref[...] = jnp.zeros_like(o_ref)

        def step(n, carry):
            bidx = batch_ref[n]
            row = h_ref[pl.ds(n, 1), :]
            cur = o_ref[pl.ds(bidx, 1), :]
            o_ref[pl.ds(bidx, 1), :] = jnp.maximum(cur, row)
            return carry

        lax.fori_loop(0, N, step, 0)

    return pl.pallas_call(
        body,
        in_specs=[
            pl.BlockSpec(memory_space=pltpu.SMEM),
            pl.BlockSpec(memory_space=pltpu.VMEM),
        ],
        out_specs=pl.BlockSpec(memory_space=pltpu.VMEM),
        out_shape=jax.ShapeDtypeStruct((B, wp), jnp.float32),
    )(batch, h)


def _head(g1, g2, cell, Wg1, bg1, Wg2, bg2, Wr1, br1, Wr2, br2, Wr3, br3,
          Wf1, bf1, Wf2, bf2, Wo, bo):
    def body(g1_ref, g2_ref, cell_ref, wg1, bg1_, wg2, bg2_, wr1, br1_,
             wr2, br2_, wr3, br3_, wf1, bf1_, wf2, bf2_, wo, bo_, o_ref):
        def dot(a, b):
            return jnp.dot(a, b, preferred_element_type=jnp.float32)

        d1 = dot(jnp.maximum(dot(g1_ref[...], wg1[...]) + bg1_[...], 0.0),
                 wg2[...]) + bg2_[...]
        d2 = dot(jnp.maximum(dot(g2_ref[...], wg1[...]) + bg1_[...], 0.0),
                 wg2[...]) + bg2_[...]
        c = cell_ref[...]
        nrm = jnp.sqrt(jnp.sum(c * c, axis=1, keepdims=True))
        cn = c / jnp.maximum(nrm, 1e-12)
        cv = jnp.maximum(dot(cn, wr1[...]) + br1_[...], 0.0)
        cv = jnp.maximum(dot(cv, wr2[...]) + br2_[...], 0.0)
        cv = dot(cv, wr3[...]) + br3_[...]
        xc = jnp.concatenate([d1, d2, cv], axis=1)
        xc = jnp.maximum(dot(xc, wf1[...]) + bf1_[...], 0.0)
        xc = jnp.maximum(dot(xc, wf2[...]) + bf2_[...], 0.0)
        o_ref[...] = dot(xc, wo[...]) + bo_[...]

    args = (g1, g2, cell, Wg1, bg1.reshape(1, -1), Wg2, bg2.reshape(1, -1),
            Wr1, br1.reshape(1, -1), Wr2, br2.reshape(1, -1), Wr3,
            br3.reshape(1, -1), Wf1, bf1.reshape(1, -1), Wf2,
            bf2.reshape(1, -1), Wo, bo.reshape(1, -1))
    return pl.pallas_call(
        body,
        out_shape=jax.ShapeDtypeStruct((B, 2), jnp.float32),
    )(*args)


def _pad_w(w, rows, cols):
    return jnp.pad(w, ((0, rows - w.shape[0]), (0, cols - w.shape[1])))


def _encoder(x, ei, batch, W1p, b1p, W2p, b2p, W3p, b3p):
    s_idx = ei[0]
    d_idx = ei[1]
    ones = jnp.ones((N, 16), jnp.float32)
    degp = _sc_scatter_add(ones, s_idx, d_idx)
    dinv = _dinv_from_deg(degp[0], degp[1])

    h = x
    for wgt, bias, wout in ((W1p, b1p, 80), (W2p, b2p, 160), (W3p, b3p, 320)):
        u = _mm_scale(h, wgt, dinv)
        if wout <= 160:
            agg = _sc_scatter_add(u, s_idx, d_idx)
            a0, a1 = agg[0], agg[1]
        else:
            aggl = _sc_scatter_add(u[:, :160], s_idx, d_idx)
            aggr = _sc_scatter_add(u[:, 160:], s_idx, d_idx)
            a0 = jnp.concatenate([aggl[0], aggr[0]], axis=1)
            a1 = jnp.concatenate([aggl[1], aggr[1]], axis=1)
        h = _combine(a0, a1, u, dinv, bias)
    return _segment_max(h, batch)


def kernel(x1, edge_index1, batch1, cell, x2, edge_index2, batch2, W1, b1,
           W2, b2, W3, b3, Wg1, bg1, Wg2, bg2, Wr1, br1, Wr2, br2, Wr3, br3,
           Wf1, bf1, Wf2, bf2, Wo, bo):
    W1p = _pad_w(W1, 78, 80)
    W2p = _pad_w(W2, 80, 160)
    W3p = _pad_w(W3, 160, 320)
    b1p = jnp.pad(b1, (0, 2)).reshape(1, -1)
    b2p = jnp.pad(b2, (0, 4)).reshape(1, -1)
    b3p = jnp.pad(b3, (0, 8)).reshape(1, -1)
    Wg1p = _pad_w(Wg1, 320, 160)
    bg1p = jnp.pad(bg1, (0, 4))
    Wg2p = _pad_w(Wg2, 160, 128)

    g1 = _encoder(x1, edge_index1, batch1, W1p, b1p, W2p, b2p, W3p, b3p)
    g2 = _encoder(x2, edge_index2, batch2, W1p, b1p, W2p, b2p, W3p, b3p)
    return _head(g1, g2, cell, Wg1p, bg1p, Wg2p, bg2, Wr1, br1, Wr2, br2,
                 Wr3, br3, Wf1, bf1, Wf2, bf2, Wo, bo)


# speedup vs baseline: 10.3257x; 1.5189x over previous
import functools

import jax
import jax.numpy as jnp
from jax import lax
from jax.experimental import pallas as pl
from jax.experimental.pallas import tpu as pltpu
from jax.experimental.pallas import tpu_sc as plsc

N = 10000
NP = 10240
E = 320000
B = 512
NC = 2
NS = 16
CH = 80


@functools.lru_cache(maxsize=None)
def _make_sc_scatter_add(n, w):
    ept = E // (NC * NS)
    iters = ept // CH
    rps = NP // NS
    mesh = plsc.VectorSubcoreMesh(core_axis_name="c", subcore_axis_name="s")

    @functools.partial(
        pl.kernel, mesh=mesh,
        compiler_params=pltpu.CompilerParams(use_tc_tiling_on_sc=False),
        out_type=jax.ShapeDtypeStruct((NC, NP, w), jnp.float32),
        scratch_types=[
            pltpu.VMEM((2, CH), jnp.int32),
            pltpu.VMEM((2, CH), jnp.int32),
            pltpu.VMEM((2, CH, w), jnp.float32),
            pltpu.VMEM_SHARED((NP, w), jnp.float32),
            pltpu.SemaphoreType.DMA,
        ],
    )
    def k(u_hbm, s_hbm, d_hbm, z_hbm, out_hbm, sv, dv, rows, acc, sem):
        cid = lax.axis_index("c")
        sid = lax.axis_index("s")
        pltpu.sync_copy(z_hbm.at[pl.ds(sid * rps, rps)],
                        acc.at[pl.ds(sid * rps, rps)])
        plsc.subcore_barrier()
        base = (cid * NS + sid) * ept

        def fetch(slot, off):
            pltpu.sync_copy(s_hbm.at[pl.ds(off, CH)], sv.at[slot])
            pltpu.sync_copy(d_hbm.at[pl.ds(off, CH)], dv.at[slot])
            pltpu.async_copy(u_hbm.at[sv.at[slot]], rows.at[slot], sem)

        fetch(0, base)

        def body(i, carry):
            cur = lax.rem(i, 2)
            nxt = 1 - cur

            @pl.when(i + 1 < iters)
            def _():
                fetch(nxt, base + (i + 1) * CH)

            pltpu.make_async_copy(
                u_hbm.at[sv.at[cur]], rows.at[cur], sem).wait()
            pltpu.sync_copy(rows.at[cur], acc.at[dv.at[cur]], add=True)
            return carry

        lax.fori_loop(0, iters, body, 0)
        plsc.subcore_barrier()
        pltpu.sync_copy(acc.at[pl.ds(sid * rps, rps)],
                        out_hbm.at[cid, pl.ds(sid * rps, rps)])

    return k


def _sc_scatter_add(u, s_idx, d_idx):
    n, w = u.shape
    zero = jnp.zeros((NP, w), jnp.float32)
    out = _make_sc_scatter_add(n, w)(u, s_idx, d_idx, zero)
    return out[:, :N, :]


_RB = 1000


def _mm_scale(h, w, dinv):
    din = h.shape[1]
    wp = w.shape[1]

    def body(h_ref, w_ref, dv_ref, o_ref):
        o_ref[...] = dv_ref[...] * jnp.dot(
            h_ref[...], w_ref[...], preferred_element_type=jnp.float32)

    return pl.pallas_call(
        body,
        grid=(N // _RB,),
        in_specs=[
            pl.BlockSpec((_RB, din), lambda i: (i, 0)),
            pl.BlockSpec((din, wp), lambda i: (0, 0)),
            pl.BlockSpec((_RB, 1), lambda i: (i, 0)),
        ],
        out_specs=pl.BlockSpec((_RB, wp), lambda i: (i, 0)),
        out_shape=jax.ShapeDtypeStruct((N, wp), jnp.float32),
    )(h, w, dinv)


def _combine(a0, a1, u, dinv, b):
    wp = u.shape[1]

    def body(a0_ref, a1_ref, u_ref, dv_ref, b_ref, o_ref):
        s = a0_ref[...] + a1_ref[...] + u_ref[...]
        o_ref[...] = jnp.maximum(dv_ref[...] * s + b_ref[...], 0.0)

    return pl.pallas_call(
        body,
        grid=(N // _RB,),
        in_specs=[
            pl.BlockSpec((_RB, wp), lambda i: (i, 0)),
            pl.BlockSpec((_RB, wp), lambda i: (i, 0)),
            pl.BlockSpec((_RB, wp), lambda i: (i, 0)),
            pl.BlockSpec((_RB, 1), lambda i: (i, 0)),
            pl.BlockSpec((1, wp), lambda i: (0, 0)),
        ],
        out_specs=pl.BlockSpec((_RB, wp), lambda i: (i, 0)),
        out_shape=jax.ShapeDtypeStruct((N, wp), jnp.float32),
    )(a0, a1, u, dinv, b)


def _dinv_from_deg(d0, d1):
    def body(d0_ref, d1_ref, o_ref):
        deg = 1.0 + d0_ref[:, 0:1] + d1_ref[:, 0:1]
        o_ref[...] = lax.rsqrt(deg)

    return pl.pallas_call(
        body,
        out_shape=jax.ShapeDtypeStruct((N, 1), jnp.float32),
    )(d0, d1)


def _segment_max(h, batch):
    wp = h.shape[1]

    def body(batch_ref, h_ref, o_ref):
        o_ref[...] = jnp.zeros_like(o_ref)

        def step(n, carry):
            bidx = batch_ref[n]
            row = h_ref[pl.ds(n, 1), :]
            cur = o_ref[pl.ds(bidx, 1), :]
            o_ref[pl.ds(bidx, 1), :] = jnp.maximum(cur, row)
            return carry

        lax.fori_loop(0, N, step, 0)

    return pl.pallas_call(
        body,
        in_specs=[
            pl.BlockSpec(memory_space=pltpu.SMEM),
            pl.BlockSpec(memory_space=pltpu.VMEM),
        ],
        out_specs=pl.BlockSpec(memory_space=pltpu.VMEM),
        out_shape=jax.ShapeDtypeStruct((B, wp), jnp.float32),
    )(batch, h)


def _head(g1, g2, cell, Wg1, bg1, Wg2, bg2, Wr1, br1, Wr2, br2, Wr3, br3,
          Wf1, bf1, Wf2, bf2, Wo, bo):
    def body(g1_ref, g2_ref, cell_ref, wg1, bg1_, wg2, bg2_, wr1, br1_,
             wr2, br2_, wr3, br3_, wf1, bf1_, wf2, bf2_, wo, bo_, o_ref):
        def dot(a, b):
            return jnp.dot(a, b, preferred_element_type=jnp.float32)

        d1 = dot(jnp.maximum(dot(g1_ref[...], wg1[...]) + bg1_[...], 0.0),
                 wg2[...]) + bg2_[...]
        d2 = dot(jnp.maximum(dot(g2_ref[...], wg1[...]) + bg1_[...], 0.0),
                 wg2[...]) + bg2_[...]
        c = cell_ref[...]
        nrm = jnp.sqrt(jnp.sum(c * c, axis=1, keepdims=True))
        cn = c / jnp.maximum(nrm, 1e-12)
        cv = jnp.maximum(dot(cn, wr1[...]) + br1_[...], 0.0)
        cv = jnp.maximum(dot(cv, wr2[...]) + br2_[...], 0.0)
        cv = dot(cv, wr3[...]) + br3_[...]
        xc = jnp.concatenate([d1, d2, cv], axis=1)
        xc = jnp.maximum(dot(xc, wf1[...]) + bf1_[...], 0.0)
        xc = jnp.maximum(dot(xc, wf2[...]) + bf2_[...], 0.0)
        o_ref[...] = dot(xc, wo[...]) + bo_[...]

    args = (g1, g2, cell, Wg1, bg1.reshape(1, -1), Wg2, bg2.reshape(1, -1),
            Wr1, br1.reshape(1, -1), Wr2, br2.reshape(1, -1), Wr3,
            br3.reshape(1, -1), Wf1, bf1.reshape(1, -1), Wf2,
            bf2.reshape(1, -1), Wo, bo.reshape(1, -1))
    return pl.pallas_call(
        body,
        out_shape=jax.ShapeDtypeStruct((B, 2), jnp.float32),
    )(*args)


def _pad_w(w, rows, cols):
    return jnp.pad(w, ((0, rows - w.shape[0]), (0, cols - w.shape[1])))


def _encoder(x, ei, batch, W1p, b1p, W2p, b2p, W3p, b3p):
    s_idx = ei[0]
    d_idx = ei[1]
    ones = jnp.ones((N, 16), jnp.float32)
    degp = _sc_scatter_add(ones, s_idx, d_idx)
    dinv = _dinv_from_deg(degp[0], degp[1])

    h = x
    for wgt, bias, wout in ((W1p, b1p, 80), (W2p, b2p, 160), (W3p, b3p, 320)):
        u = _mm_scale(h, wgt, dinv)
        if wout <= 160:
            agg = _sc_scatter_add(u, s_idx, d_idx)
            a0, a1 = agg[0], agg[1]
        else:
            aggl = _sc_scatter_add(u[:, :160], s_idx, d_idx)
            aggr = _sc_scatter_add(u[:, 160:], s_idx, d_idx)
            a0 = jnp.concatenate([aggl[0], aggr[0]], axis=1)
            a1 = jnp.concatenate([aggl[1], aggr[1]], axis=1)
        h = _combine(a0, a1, u, dinv, bias)
    return _segment_max(h, batch)


def kernel(x1, edge_index1, batch1, cell, x2, edge_index2, batch2, W1, b1,
           W2, b2, W3, b3, Wg1, bg1, Wg2, bg2, Wr1, br1, Wr2, br2, Wr3, br3,
           Wf1, bf1, Wf2, bf2, Wo, bo):
    W1p = _pad_w(W1, 78, 80)
    W2p = _pad_w(W2, 80, 160)
    W3p = _pad_w(W3, 160, 320)
    b1p = jnp.pad(b1, (0, 2)).reshape(1, -1)
    b2p = jnp.pad(b2, (0, 4)).reshape(1, -1)
    b3p = jnp.pad(b3, (0, 8)).reshape(1, -1)
    Wg1p = _pad_w(Wg1, 320, 160)
    bg1p = jnp.pad(bg1, (0, 4))
    Wg2p = _pad_w(Wg2, 160, 128)

    g1 = _encoder(x1, edge_index1, batch1, W1p, b1p, W2p, b2p, W3p, b3p)
    g2 = _encoder(x2, edge_index2, batch2, W1p, b1p, W2p, b2p, W3p, b3p)
    return _head(g1, g2, cell, Wg1p, bg1p, Wg2p, bg2, Wr1, br1, Wr2, br2,
                 Wr3, br3, Wf1, bf1, Wf2, bf2, Wo, bo)
